# baseline (speedup 1.0000x reference)
import functools

import jax
import jax.numpy as jnp
from jax import lax
from jax.experimental import pallas as pl
from jax.experimental.pallas import tpu as pltpu
from jax.experimental.pallas import tpu_sc as plsc

DIM_VOCAB = 100000
DIM_HIDDEN = 64
PAD_DIM = 128
BATCH = 4096
HIST_LEN = 50

NUM_WORKERS = 32
B_PER_W = BATCH // NUM_WORKERS
NBUF = 2
N_ROUNDS = HIST_LEN // NBUF

_mesh = plsc.VectorSubcoreMesh(core_axis_name="c", subcore_axis_name="s")


@functools.partial(
    pl.kernel,
    out_type=jax.ShapeDtypeStruct((HIST_LEN, DIM_HIDDEN, BATCH), jnp.float32),
    mesh=_mesh,
    scratch_types=[
        pltpu.VMEM((HIST_LEN, B_PER_W), jnp.int32),
        [pltpu.VMEM((B_PER_W, PAD_DIM), jnp.float32) for _ in range(NBUF)],
        [pltpu.VMEM((DIM_HIDDEN, B_PER_W), jnp.float32) for _ in range(NBUF)],
        [pltpu.SemaphoreType.DMA for _ in range(NBUF)],
        [pltpu.SemaphoreType.DMA for _ in range(NBUF)],
    ],
    compiler_params=pltpu.CompilerParams(
        use_tc_tiling_on_sc=True, needs_layout_passes=False),
)
def _embed_lookup(idx_hbm, table_hbm, out_hbm, idx_v, rows, tr, gsem, osem):
    wid = lax.axis_index("s") * 2 + lax.axis_index("c")
    bbase = wid * B_PER_W
    pltpu.sync_copy(idx_hbm.at[:, pl.ds(bbase, B_PER_W)], idx_v)

    def gather(t, b):
        return pltpu.make_async_copy(
            table_hbm.at[idx_v.at[t]], rows[b], gsem[b])

    def put(t, b):
        return pltpu.make_async_copy(
            tr[b], out_hbm.at[t, :, pl.ds(bbase, B_PER_W)], osem[b])

    iota16 = lax.iota(jnp.int32, 16)

    def transpose(b):
        def quad(j, carry):
            for u in range(4):
                c = j * 4 + u
                cvec = jnp.full((16,), c, jnp.int32)
                for blk in range(B_PER_W // 16):
                    v = plsc.load_gather(rows[b], [iota16 + blk * 16, cvec])
                    tr[b][c, pl.ds(blk * 16, 16)] = v
            return carry
        lax.fori_loop(0, DIM_HIDDEN // 4, quad, 0)

    def body(g, carry):
        for b in range(NBUF):
            t = g * NBUF + b
            @pl.when(g > 0)
            def _():
                put(t - NBUF, b).wait()
            gather(t, b).start()
        for b in range(NBUF):
            t = g * NBUF + b
            gather(t, b).wait()
            transpose(b)
            put(t, b).start()
        return carry

    lax.fori_loop(0, N_ROUNDS, body, 0)
    for b in range(NBUF):
        put(HIST_LEN - NBUF + b, b).wait()


def kernel(x, emb_t):
    x_t = jnp.swapaxes(x, 0, 1).astype(jnp.int32)
    table = jnp.pad(emb_t, ((0, 0), (0, PAD_DIM - DIM_HIDDEN)))
    out_t = _embed_lookup(x_t, table)
    return jnp.transpose(out_t, (2, 0, 1))


# speedup vs baseline: 1.1693x; 1.1693x over previous
import functools

import jax
import jax.numpy as jnp
from jax import lax
from jax.experimental import pallas as pl
from jax.experimental.pallas import tpu as pltpu
from jax.experimental.pallas import tpu_sc as plsc

DIM_VOCAB = 100000
DIM_HIDDEN = 64
PAD_DIM = 128
BATCH = 4096
HIST_LEN = 50

NUM_WORKERS = 32
B_PER_W = BATCH // NUM_WORKERS
NBUF = 2
N_ROUNDS = HIST_LEN // NBUF

_mesh = plsc.VectorSubcoreMesh(core_axis_name="c", subcore_axis_name="s")


@functools.partial(
    pl.kernel,
    out_type=jax.ShapeDtypeStruct((HIST_LEN, DIM_HIDDEN, BATCH), jnp.float32),
    mesh=_mesh,
    scratch_types=[
        pltpu.VMEM((HIST_LEN, B_PER_W), jnp.int32),
        [pltpu.VMEM((B_PER_W, PAD_DIM), jnp.float32) for _ in range(NBUF)],
        [pltpu.VMEM((DIM_HIDDEN, B_PER_W), jnp.float32) for _ in range(NBUF)],
        [pltpu.SemaphoreType.DMA for _ in range(NBUF)],
        [pltpu.SemaphoreType.DMA for _ in range(NBUF)],
    ],
    compiler_params=pltpu.CompilerParams(
        use_tc_tiling_on_sc=True, needs_layout_passes=False),
)
def _embed_lookup(idx_hbm, table_hbm, out_hbm, idx_v, rows, tr, gsem, osem):
    wid = lax.axis_index("s") * 2 + lax.axis_index("c")
    bbase = wid * B_PER_W
    pltpu.sync_copy(idx_hbm.at[:, pl.ds(bbase, B_PER_W)], idx_v)

    def gather(t, b):
        return pltpu.make_async_copy(
            table_hbm.at[idx_v.at[t]], rows[b], gsem[b])

    def put(t, b):
        return pltpu.make_async_copy(
            tr[b], out_hbm.at[t, :, pl.ds(bbase, B_PER_W)], osem[b])

    iota16 = lax.iota(jnp.int32, 16)
    ridx = [iota16 + blk * 16 for blk in range(B_PER_W // 16)]

    def transpose(b):
        def col(c, carry):
            cvec = jnp.full((16,), c, jnp.int32)
            vs = [plsc.load_gather(rows[b], [ridx[blk], cvec])
                  for blk in range(B_PER_W // 16)]
            for blk in range(B_PER_W // 16):
                tr[b][c, pl.ds(blk * 16, 16)] = vs[blk]
            return carry
        lax.fori_loop(0, DIM_HIDDEN, col, 0)

    def body(g, carry):
        for b in range(NBUF):
            t = g * NBUF + b
            @pl.when(g > 0)
            def _():
                put(t - NBUF, b).wait()
            gather(t, b).start()
        for b in range(NBUF):
            t = g * NBUF + b
            gather(t, b).wait()
            transpose(b)
            put(t, b).start()
        return carry

    lax.fori_loop(0, N_ROUNDS, body, 0)
    for b in range(NBUF):
        put(HIST_LEN - NBUF + b, b).wait()


def kernel(x, emb_t):
    x_t = jnp.swapaxes(x, 0, 1).astype(jnp.int32)
    table = jnp.pad(emb_t, ((0, 0), (0, PAD_DIM - DIM_HIDDEN)))
    out_t = _embed_lookup(x_t, table)
    return jnp.transpose(out_t, (2, 0, 1))


# speedup vs baseline: 1.8309x; 1.5658x over previous
import functools

import jax
import jax.numpy as jnp
from jax import lax
from jax.experimental import pallas as pl
from jax.experimental.pallas import tpu as pltpu
from jax.experimental.pallas import tpu_sc as plsc

DIM_VOCAB = 100000
DIM_HIDDEN = 64
PAD_DIM = 128
BATCH = 4096
HIST_LEN = 50

NUM_WORKERS = 32
B_PER_W = BATCH // NUM_WORKERS
NBUF = 4
N_ROUNDS = B_PER_W // NBUF

_mesh = plsc.VectorSubcoreMesh(core_axis_name="c", subcore_axis_name="s")


@functools.partial(
    pl.kernel,
    out_type=jax.ShapeDtypeStruct((BATCH, HIST_LEN, DIM_HIDDEN), jnp.float32),
    mesh=_mesh,
    scratch_types=[
        pltpu.VMEM((B_PER_W, HIST_LEN), jnp.int32),
        [pltpu.VMEM((HIST_LEN, PAD_DIM), jnp.float32) for _ in range(NBUF)],
        [pltpu.VMEM((HIST_LEN, DIM_HIDDEN), jnp.float32) for _ in range(NBUF)],
        [pltpu.SemaphoreType.DMA for _ in range(NBUF)],
        [pltpu.SemaphoreType.DMA for _ in range(NBUF)],
    ],
    compiler_params=pltpu.CompilerParams(use_tc_tiling_on_sc=True),
)
def _embed_lookup(idx_hbm, table_hbm, out_hbm, idx_v, rows, pk, gsem, osem):
    wid = lax.axis_index("s") * 2 + lax.axis_index("c")
    base = wid * B_PER_W
    pltpu.sync_copy(idx_hbm.at[pl.ds(base, B_PER_W)], idx_v)

    def gather(c, b):
        return pltpu.make_async_copy(
            table_hbm.at[idx_v.at[c]], rows[b], gsem[b])

    def put(c, b):
        return pltpu.make_async_copy(pk[b], out_hbm.at[base + c], osem[b])

    def repack(b):
        def row(t, carry):
            for k in range(DIM_HIDDEN // 16):
                pk[b][t, pl.ds(k * 16, 16)] = rows[b][t, pl.ds(k * 16, 16)]
            return carry
        lax.fori_loop(0, HIST_LEN, row, 0)

    def body(g, carry):
        for b in range(NBUF):
            c = g * NBUF + b
            @pl.when(g > 0)
            def _():
                put(c - NBUF, b).wait()
            gather(c, b).start()
        for b in range(NBUF):
            c = g * NBUF + b
            gather(c, b).wait()
            repack(b)
            put(c, b).start()
        return carry

    lax.fori_loop(0, N_ROUNDS, body, 0)
    for b in range(NBUF):
        put(B_PER_W - NBUF + b, b).wait()


def kernel(x, emb_t):
    table = jnp.pad(emb_t, ((0, 0), (0, PAD_DIM - DIM_HIDDEN)))
    return _embed_lookup(x.astype(jnp.int32), table)


# speedup vs baseline: 1.8399x; 1.0049x over previous
import functools

import jax
import jax.numpy as jnp
from jax import lax
from jax.experimental import pallas as pl
from jax.experimental.pallas import tpu as pltpu
from jax.experimental.pallas import tpu_sc as plsc

DIM_VOCAB = 100000
DIM_HIDDEN = 64
PAD_DIM = 128
BATCH = 4096
HIST_LEN = 50

NUM_WORKERS = 32
B_PER_W = BATCH // NUM_WORKERS
NBUF = 4
N_ROUNDS = B_PER_W // NBUF

_mesh = plsc.VectorSubcoreMesh(core_axis_name="c", subcore_axis_name="s")


@functools.partial(
    pl.kernel,
    out_type=jax.ShapeDtypeStruct((BATCH, HIST_LEN, DIM_HIDDEN), jnp.float32),
    mesh=_mesh,
    scratch_types=[
        pltpu.VMEM((B_PER_W, HIST_LEN), jnp.int32),
        [pltpu.VMEM((HIST_LEN, PAD_DIM), jnp.float32) for _ in range(NBUF)],
        [pltpu.VMEM((HIST_LEN, DIM_HIDDEN), jnp.float32) for _ in range(NBUF)],
        [pltpu.SemaphoreType.DMA for _ in range(NBUF)],
        [pltpu.SemaphoreType.DMA for _ in range(NBUF)],
    ],
    compiler_params=pltpu.CompilerParams(use_tc_tiling_on_sc=True),
)
def _embed_lookup(idx_hbm, table_hbm, out_hbm, idx_v, rows, pk, gsem, osem):
    wid = lax.axis_index("s") * 2 + lax.axis_index("c")
    base = wid * B_PER_W
    pltpu.sync_copy(idx_hbm.at[pl.ds(base, B_PER_W)], idx_v)

    def gather(c, b):
        return pltpu.make_async_copy(
            table_hbm.at[idx_v.at[c]], rows[b], gsem[b])

    def put(c, b):
        return pltpu.make_async_copy(pk[b], out_hbm.at[base + c], osem[b])

    def repack(b):
        def row(i, carry):
            t = i * 2
            vs = [rows[b][t + u, pl.ds(k * 16, 16)]
                  for u in range(2) for k in range(DIM_HIDDEN // 16)]
            for u in range(2):
                for k in range(DIM_HIDDEN // 16):
                    pk[b][t + u, pl.ds(k * 16, 16)] = vs[u * 4 + k]
            return carry
        lax.fori_loop(0, HIST_LEN // 2, row, 0)

    def body(g, carry):
        for b in range(NBUF):
            c = g * NBUF + b
            @pl.when(g > 0)
            def _():
                put(c - NBUF, b).wait()
            gather(c, b).start()
        for b in range(NBUF):
            c = g * NBUF + b
            gather(c, b).wait()
            repack(b)
            put(c, b).start()
        return carry

    lax.fori_loop(0, N_ROUNDS, body, 0)
    for b in range(NBUF):
        put(B_PER_W - NBUF + b, b).wait()


def kernel(x, emb_t):
    table = jnp.pad(emb_t, ((0, 0), (0, PAD_DIM - DIM_HIDDEN)))
    return _embed_lookup(x.astype(jnp.int32), table)
